# baseline (speedup 1.0000x reference)
import functools

import jax
import jax.numpy as jnp
from jax import lax
from jax.experimental import pallas as pl
from jax.experimental.pallas import tpu as pltpu
from jax.experimental.pallas import tpu_sc as plsc

N = 2048
E = 65536
HID = 128
NHEADS = 16
DE = 16
DEH = 4
ALPHA = 0.2
D2H = 2 * HID
F1 = NHEADS * D2H
DE1 = NHEADS * DEH
N1 = N // 2

BS1 = 256
NBLK1 = N // BS1
BS2 = 256
NBLK2 = N1 // BS2

_f32 = jnp.float32


def _leaky(x):
    return jnp.where(x >= 0, x, ALPHA * x)


def _elu(x):
    return jnp.where(x > 0, x, jnp.exp(jnp.minimum(x, 0.0)) - 1.0)


NN = N * N
_SC_NC = 2
_SC_NS = 16
_NW = _SC_NC * _SC_NS
_CE = E // _NW
_i32 = jnp.int32


def _sc_gather_body(ei_hbm, n2n_hbm, g_hbm, lin_hbm,
                    src_v, dst_v, idx_v, rows_v, sem):
    wid = lax.axis_index("s") * _SC_NC + lax.axis_index("c")
    base = wid * _CE
    pltpu.sync_copy(ei_hbm.at[0, pl.ds(base, _CE)], src_v)
    pltpu.sync_copy(ei_hbm.at[1, pl.ds(base, _CE)], dst_v)

    def body(j, carry):
        sl = pl.ds(j * 16, 16)
        idx_v[sl] = src_v[sl] * N + dst_v[sl]
        return carry

    lax.fori_loop(0, _CE // 16, body, 0)
    pltpu.sync_copy(idx_v, lin_hbm.at[pl.ds(base, _CE)])
    pltpu.async_copy(n2n_hbm.at[idx_v], rows_v, sem).wait()
    pltpu.sync_copy(rows_v, g_hbm.at[pl.ds(base, _CE)])


def _sc_gather(edge_index, node2node):
    run = pl.kernel(
        _sc_gather_body,
        mesh=plsc.VectorSubcoreMesh(core_axis_name="c", subcore_axis_name="s"),
        compiler_params=pltpu.CompilerParams(use_tc_tiling_on_sc=False),
        out_type=[
            jax.ShapeDtypeStruct((E, DE), _f32),
            jax.ShapeDtypeStruct((E,), _i32),
        ],
        scratch_types=[
            pltpu.VMEM((_CE,), _i32),
            pltpu.VMEM((_CE,), _i32),
            pltpu.VMEM((_CE,), _i32),
            pltpu.VMEM((_CE, DE), _f32),
            pltpu.SemaphoreType.DMA,
        ],
    )
    return run(edge_index, node2node)


def _sc_scatter_body(lin_hbm, fea_hbm, out_hbm, lin_v, val_v, sem):
    wid = lax.axis_index("s") * _SC_NC + lax.axis_index("c")
    base = wid * _CE
    pltpu.sync_copy(lin_hbm.at[pl.ds(base, _CE)], lin_v)
    pltpu.sync_copy(fea_hbm.at[pl.ds(base, _CE)], val_v)
    pltpu.async_copy(val_v, out_hbm.at[lin_v], sem).wait()


def _sc_scatter(lin, fe_all):
    run = pl.kernel(
        _sc_scatter_body,
        mesh=plsc.VectorSubcoreMesh(core_axis_name="c", subcore_axis_name="s"),
        compiler_params=pltpu.CompilerParams(use_tc_tiling_on_sc=False),
        out_type=jax.ShapeDtypeStruct((NN, NHEADS), _f32),
        scratch_types=[
            pltpu.VMEM((_CE,), _i32),
            pltpu.VMEM((_CE, NHEADS), _f32),
            pltpu.SemaphoreType.DMA,
        ],
    )
    return run(lin, fe_all)


_SBD = 16


def _fed_body(a3_ref, n2n_ref, o_ref):
    v3 = jnp.reshape(n2n_ref[...], (_SBD * N // 128, 128, DE))
    t = lax.dot_general(a3_ref[...], v3,
                        (((1,), (2,)), ((), ())),
                        preferred_element_type=_f32)
    o_ref[...] = jnp.reshape(t, (NHEADS, _SBD, N))


def _fe_dense(a3, n2n_wide):
    return pl.pallas_call(
        _fed_body,
        grid=(N // _SBD,),
        in_specs=[
            pl.BlockSpec((NHEADS, DE), lambda sb: (0, 0)),
            pl.BlockSpec((_SBD * N // 128, 128 * DE), lambda sb: (sb, 0)),
        ],
        out_specs=pl.BlockSpec((NHEADS, _SBD, N), lambda sb: (0, sb, 0)),
        out_shape=jax.ShapeDtypeStruct((NHEADS, N, N), _f32),
    )(a3, n2n_wide)


def _fet_body(a3_ref, g_ref, o_ref):
    o_ref[...] = lax.dot_general(g_ref[...], a3_ref[...],
                                 (((1,), (1,)), ((), ())),
                                 preferred_element_type=_f32)


def _fet(a3, g):
    be = 8192
    return pl.pallas_call(
        _fet_body,
        grid=(E // be,),
        in_specs=[
            pl.BlockSpec((NHEADS, DE), lambda b: (0, 0)),
            pl.BlockSpec((be, DE), lambda b: (b, 0)),
        ],
        out_specs=pl.BlockSpec((be, NHEADS), lambda b: (b, 0)),
        out_shape=jax.ShapeDtypeStruct((E, NHEADS), _f32),
    )(a3, g)


def _prep_body(f_ref, wh_ref, ah_ref, g1w_ref, g1b_ref, wg_ref, a1_ref, a2_ref,
               h_ref, hs0_ref, f1t_ref, f2t_ref):
    f = f_ref[...]
    z = jnp.dot(f, wh_ref[...], preferred_element_type=_f32)
    gate = jax.nn.sigmoid(jnp.dot(f, ah_ref[...], preferred_element_type=_f32))
    h = _elu(z * gate)
    h_ref[...] = h
    g = jax.nn.sigmoid(jnp.dot(h, g1w_ref[...], preferred_element_type=_f32)
                       + g1b_ref[0, 0])
    g = g - jnp.max(g, axis=0, keepdims=True)
    p = jnp.exp(g)
    att0 = p / jnp.sum(p, axis=0, keepdims=True)
    hs0_ref[...] = lax.dot_general(att0, h, (((0,), (0,)), ((), ())),
                                   preferred_element_type=_f32)
    q1 = jnp.sum(wg_ref[...] * a1_ref[...][:, None, :], axis=-1)
    q2 = jnp.sum(wg_ref[...] * a2_ref[...][:, None, :], axis=-1)
    f1t_ref[...] = lax.dot_general(q1, h, (((1,), (1,)), ((), ())),
                                   preferred_element_type=_f32)
    f2t_ref[...] = lax.dot_general(q2, h, (((1,), (1,)), ((), ())),
                                   preferred_element_type=_f32)


def _prep(features, params):
    return pl.pallas_call(
        _prep_body,
        out_shape=[
            jax.ShapeDtypeStruct((N, HID), _f32),
            jax.ShapeDtypeStruct((1, HID), _f32),
            jax.ShapeDtypeStruct((NHEADS, N), _f32),
            jax.ShapeDtypeStruct((NHEADS, N), _f32),
        ],
    )(features, params['W_h'], params['a_h'], params['g1_w'],
      params['g1_b'].reshape(1, 1), params['Wg'], params['a1'], params['a2'])


def _attn1_body(h_ref, wg_ref, f1_ref, f2_ref, adj_ref, fe_ref, out_ref, whs):
    rb = pl.program_id(0)
    i = pl.program_id(1)

    @pl.when((rb == 0) & (i == 0))
    def _():
        hv = h_ref[...]
        for hh in range(NHEADS):
            whs[hh] = jnp.dot(hv, wg_ref[hh],
                              preferred_element_type=_f32)

    f1col = jnp.reshape(f1_ref[...], (BS1, 1))
    f2row = jnp.reshape(f2_ref[...], (1, N))
    z = f1col + f2row + fe_ref[0]
    z = jnp.where(adj_ref[...] > 0, _leaky(z), -1e9)
    z = z - jnp.max(z, axis=1, keepdims=True)
    p = jnp.exp(z)
    att = p / jnp.sum(p, axis=1, keepdims=True)
    wh_i = jnp.reshape(whs[pl.ds(i, 1)], (N, D2H))
    hp = jnp.dot(att, wh_i, preferred_element_type=_f32)
    out_ref[...] = _elu(hp)


def _attn1(h, wg, f1t, f2t, adjacency, fe):
    f1r = f1t.reshape(NHEADS, NBLK1, 1, BS1)
    f2r = f2t.reshape(NHEADS, 1, N)
    return pl.pallas_call(
        _attn1_body,
        grid=(NBLK1, NHEADS),
        in_specs=[
            pl.BlockSpec((N, HID), lambda rb, i: (0, 0)),
            pl.BlockSpec((NHEADS, HID, D2H), lambda rb, i: (0, 0, 0)),
            pl.BlockSpec((1, 1, 1, BS1), lambda rb, i: (i, rb, 0, 0)),
            pl.BlockSpec((1, 1, N), lambda rb, i: (i, 0, 0)),
            pl.BlockSpec((BS1, N), lambda rb, i: (rb, 0)),
            pl.BlockSpec((1, BS1, N), lambda rb, i: (i, rb, 0)),
        ],
        out_specs=pl.BlockSpec((BS1, D2H), lambda rb, i: (rb, i)),
        out_shape=jax.ShapeDtypeStruct((N, F1), _f32),
        scratch_shapes=[pltpu.VMEM((NHEADS, N, D2H), _f32)],
    )(h, wg, f1r, f2r, adjacency, fe)


def _mm_body(a_ref, b_ref, o_ref):
    k = pl.program_id(0)

    @pl.when(k == 0)
    def _():
        o_ref[...] = jnp.zeros_like(o_ref)

    o_ref[...] += jnp.dot(a_ref[...], b_ref[...], preferred_element_type=_f32)


def _matmul(a, b, bk):
    m, k = a.shape
    _, n = b.shape
    return pl.pallas_call(
        _mm_body,
        grid=(k // bk,),
        in_specs=[
            pl.BlockSpec((m, bk), lambda kb: (0, kb)),
            pl.BlockSpec((bk, n), lambda kb: (kb, 0)),
        ],
        out_specs=pl.BlockSpec((m, n), lambda kb: (0, 0)),
        out_shape=jax.ShapeDtypeStruct((m, n), _f32),
    )(a, b)


def _attn2_body(wh2_ref, wh2b_ref, ao1_ref, ao2_ref, adj_ref, fe_ref, out_ref):
    wh2 = wh2_ref[...]
    f2row = lax.dot_general(ao2_ref[...], wh2, (((1,), (1,)), ((), ())),
                            preferred_element_type=_f32)
    f1col = lax.dot_general(wh2b_ref[...], ao1_ref[...], (((1,), (1,)), ((), ())),
                            preferred_element_type=_f32)
    z = f1col + f2row + fe_ref[...]
    z = jnp.where(adj_ref[...] > 0, _leaky(z), -1e9)
    z = z - jnp.max(z, axis=1, keepdims=True)
    p = jnp.exp(z)
    att = p / jnp.sum(p, axis=1, keepdims=True)
    out_ref[...] = jnp.dot(att, wh2, preferred_element_type=_f32)


def _attn2(wh2, ao1, ao2, adj1, fe2):
    return pl.pallas_call(
        _attn2_body,
        grid=(NBLK2,),
        in_specs=[
            pl.BlockSpec((N1, D2H), lambda rb: (0, 0)),
            pl.BlockSpec((BS2, D2H), lambda rb: (rb, 0)),
            pl.BlockSpec((1, D2H), lambda rb: (0, 0)),
            pl.BlockSpec((1, D2H), lambda rb: (0, 0)),
            pl.BlockSpec((BS2, N1), lambda rb: (rb, 0)),
            pl.BlockSpec((BS2, N1), lambda rb: (rb, 0)),
        ],
        out_specs=pl.BlockSpec((BS2, D2H), lambda rb: (rb, 0)),
        out_shape=jax.ShapeDtypeStruct((N1, D2H), _f32),
    )(wh2, wh2, ao1.reshape(1, D2H), ao2.reshape(1, D2H), adj1, fe2)


@jax.jit
def kernel(features, edge_index, edge_attr, adjacency, node2node, params):
    src, dst = edge_index[0], edge_index[1]
    lin = src * N + dst

    h, hs0, f1t, f2t = _prep(features, params)

    n2n_wide = node2node.reshape(NN // 128, 128 * DE)
    fe = _fe_dense(params['a3'], n2n_wide)

    h1 = _attn1(h, params['Wg'], f1t, f2t, adjacency, fe)

    ew = params['ep1_w']
    u = h1 @ ew[:F1, 0]
    v = h1 @ ew[F1:, 0]
    score = jax.nn.sigmoid(u[src] + v[dst] + params['ep1_b'][0])
    ngate = jax.nn.sigmoid(h1 @ params['ep1_ng_w'] + params['ep1_ng_b'])
    x1 = (h1 * ngate).reshape(N1, 2, F1).sum(axis=1)

    g2 = jax.nn.sigmoid(x1 @ params['g2_w'] + params['g2_b'])
    g2 = jax.nn.softmax(g2, axis=0)
    hs1 = jnp.sum(g2 * x1, axis=0, keepdims=True)

    we_cat = jnp.transpose(params['We'], (1, 0, 2)).reshape(DE, DE1)
    t = (edge_attr @ (we_cat @ params['ao3'])) * score
    s1, d1 = src // 2, dst // 2
    mask1 = s1 != d1
    flat1 = jnp.where(mask1, s1 * N1 + d1, N1 * N1)
    fe2 = jnp.zeros((N1 * N1 + 1,), _f32).at[flat1].set(t)[:N1 * N1]
    fe2 = fe2.reshape(N1, N1)
    adj1 = jnp.zeros((N1, N1), _f32).at[s1, d1].add(mask1.astype(_f32))

    wh2 = _matmul(x1, params['Wo'], 1024)
    h2 = _attn2(wh2, params['ao1'], params['ao2'], adj1, fe2)

    ngate2 = jax.nn.sigmoid(h2 @ params['ep2_ng_w'] + params['ep2_ng_b'])
    x2 = (h2 * ngate2).reshape(N1 // 2, 2, D2H).sum(axis=1)
    g3 = jax.nn.sigmoid(x2 @ params['g3_w'] + params['g3_b'])
    g3 = jax.nn.softmax(g3, axis=0)
    hs2 = jnp.sum(g3 * x2, axis=0, keepdims=True)

    return jnp.concatenate([hs0, hs1, hs2], axis=1)


# speedup vs baseline: 1.8585x; 1.8585x over previous
import functools

import jax
import jax.numpy as jnp
from jax import lax
from jax.experimental import pallas as pl
from jax.experimental.pallas import tpu as pltpu
from jax.experimental.pallas import tpu_sc as plsc

N = 2048
E = 65536
HID = 128
NHEADS = 16
DE = 16
DEH = 4
ALPHA = 0.2
D2H = 2 * HID
F1 = NHEADS * D2H
DE1 = NHEADS * DEH
N1 = N // 2

BS1 = 256
NBLK1 = N // BS1
BS2 = 256
NBLK2 = N1 // BS2

_f32 = jnp.float32


def _leaky(x):
    return jnp.where(x >= 0, x, ALPHA * x)


def _elu(x):
    return jnp.where(x > 0, x, jnp.exp(jnp.minimum(x, 0.0)) - 1.0)


NN = N * N
_SC_NC = 2
_SC_NS = 16
_NW = _SC_NC * _SC_NS
_CE = E // _NW
_i32 = jnp.int32


def _sc_gather_body(ei_hbm, n2n_hbm, g_hbm, lin_hbm,
                    src_v, dst_v, idx_v, rows_v, sem):
    wid = lax.axis_index("s") * _SC_NC + lax.axis_index("c")
    base = wid * _CE
    pltpu.sync_copy(ei_hbm.at[0, pl.ds(base, _CE)], src_v)
    pltpu.sync_copy(ei_hbm.at[1, pl.ds(base, _CE)], dst_v)

    def body(j, carry):
        sl = pl.ds(j * 16, 16)
        idx_v[sl] = src_v[sl] * N + dst_v[sl]
        return carry

    lax.fori_loop(0, _CE // 16, body, 0)
    pltpu.sync_copy(idx_v, lin_hbm.at[pl.ds(base, _CE)])
    pltpu.async_copy(n2n_hbm.at[idx_v], rows_v, sem).wait()
    pltpu.sync_copy(rows_v, g_hbm.at[pl.ds(base, _CE)])


def _sc_gather(edge_index, node2node):
    run = pl.kernel(
        _sc_gather_body,
        mesh=plsc.VectorSubcoreMesh(core_axis_name="c", subcore_axis_name="s"),
        compiler_params=pltpu.CompilerParams(use_tc_tiling_on_sc=False),
        out_type=[
            jax.ShapeDtypeStruct((E, DE), _f32),
            jax.ShapeDtypeStruct((E,), _i32),
        ],
        scratch_types=[
            pltpu.VMEM((_CE,), _i32),
            pltpu.VMEM((_CE,), _i32),
            pltpu.VMEM((_CE,), _i32),
            pltpu.VMEM((_CE, DE), _f32),
            pltpu.SemaphoreType.DMA,
        ],
    )
    return run(edge_index, node2node)


def _sc_scatter_body(lin_hbm, fea_hbm, out_hbm, lin_v, val_v, sem):
    wid = lax.axis_index("s") * _SC_NC + lax.axis_index("c")
    base = wid * _CE
    pltpu.sync_copy(lin_hbm.at[pl.ds(base, _CE)], lin_v)
    pltpu.sync_copy(fea_hbm.at[pl.ds(base, _CE)], val_v)
    pltpu.async_copy(val_v, out_hbm.at[lin_v], sem).wait()


def _sc_scatter(lin, fe_all):
    run = pl.kernel(
        _sc_scatter_body,
        mesh=plsc.VectorSubcoreMesh(core_axis_name="c", subcore_axis_name="s"),
        compiler_params=pltpu.CompilerParams(use_tc_tiling_on_sc=False),
        out_type=jax.ShapeDtypeStruct((NN, NHEADS), _f32),
        scratch_types=[
            pltpu.VMEM((_CE,), _i32),
            pltpu.VMEM((_CE, NHEADS), _f32),
            pltpu.SemaphoreType.DMA,
        ],
    )
    return run(lin, fe_all)


_SBD = 16


def _fed_body(a3_ref, n2n_ref, o_ref):
    t = lax.dot_general(a3_ref[...], n2n_ref[...],
                        (((1,), (1,)), ((), ())),
                        preferred_element_type=_f32)
    o_ref[...] = jnp.reshape(t, (NHEADS, _SBD, N))


def _fe_dense(a3, n2n):
    return pl.pallas_call(
        _fed_body,
        grid=(N // _SBD,),
        in_specs=[
            pl.BlockSpec((NHEADS, DE), lambda sb: (0, 0)),
            pl.BlockSpec((_SBD * N, DE), lambda sb: (sb, 0)),
        ],
        out_specs=pl.BlockSpec((NHEADS, _SBD, N), lambda sb: (0, sb, 0)),
        out_shape=jax.ShapeDtypeStruct((NHEADS, N, N), _f32),
    )(a3, n2n)


def _sc_edge_body(ei_hbm, u_hbm, v_hbm, tp_hbm, out_hbm,
                  src_v, dst_v, u_v, v_v, t_v, idx_v, val_v, sem):
    wid = lax.axis_index("s") * _SC_NC + lax.axis_index("c")
    base = wid * _CE
    pltpu.sync_copy(ei_hbm.at[0, pl.ds(base, _CE)], src_v)
    pltpu.sync_copy(ei_hbm.at[1, pl.ds(base, _CE)], dst_v)
    pltpu.sync_copy(u_hbm, u_v)
    pltpu.sync_copy(v_hbm, v_v)
    pltpu.sync_copy(tp_hbm.at[pl.ds(base, _CE)], t_v)

    def body(j, carry):
        sl = pl.ds(j * 16, 16)
        s = src_v[sl]
        d = dst_v[sl]
        ue = plsc.load_gather(u_v, [s])
        ve = plsc.load_gather(v_v, [d])
        score = 1.0 / (1.0 + jnp.exp(-(ue + ve)))
        s1 = lax.shift_right_logical(s, 1)
        d1 = lax.shift_right_logical(d, 1)
        idx_v[sl] = jnp.where(s1 != d1, s1 * N1 + d1, N1 * N1)
        val_v[sl] = score * t_v[sl]
        return carry

    lax.fori_loop(0, _CE // 16, body, 0)
    pltpu.async_copy(val_v, out_hbm.at[idx_v], sem).wait()


def _sc_edge(edge_index, u, v, tpre):
    run = pl.kernel(
        _sc_edge_body,
        mesh=plsc.VectorSubcoreMesh(core_axis_name="c", subcore_axis_name="s"),
        compiler_params=pltpu.CompilerParams(use_tc_tiling_on_sc=False,
                                             needs_layout_passes=False),
        out_type=jax.ShapeDtypeStruct((N1 * N1 + 8,), _f32),
        scratch_types=[
            pltpu.VMEM((_CE,), _i32),
            pltpu.VMEM((_CE,), _i32),
            pltpu.VMEM((N,), _f32),
            pltpu.VMEM((N,), _f32),
            pltpu.VMEM((_CE,), _f32),
            pltpu.VMEM((_CE,), _i32),
            pltpu.VMEM((_CE,), _f32),
            pltpu.SemaphoreType.DMA,
        ],
    )
    return run(edge_index, u, v, tpre)


def _fet_body(a3_ref, g_ref, o_ref):
    o_ref[...] = lax.dot_general(g_ref[...], a3_ref[...],
                                 (((1,), (1,)), ((), ())),
                                 preferred_element_type=_f32)


def _fet(a3, g):
    be = 8192
    return pl.pallas_call(
        _fet_body,
        grid=(E // be,),
        in_specs=[
            pl.BlockSpec((NHEADS, DE), lambda b: (0, 0)),
            pl.BlockSpec((be, DE), lambda b: (b, 0)),
        ],
        out_specs=pl.BlockSpec((be, NHEADS), lambda b: (b, 0)),
        out_shape=jax.ShapeDtypeStruct((E, NHEADS), _f32),
    )(a3, g)


def _prep_body(f_ref, wh_ref, ah_ref, g1w_ref, g1b_ref, wg_ref, a1_ref, a2_ref,
               h_ref, hs0_ref, f1t_ref, f2t_ref):
    f = f_ref[...]
    z = jnp.dot(f, wh_ref[...], preferred_element_type=_f32)
    gate = jax.nn.sigmoid(jnp.dot(f, ah_ref[...], preferred_element_type=_f32))
    h = _elu(z * gate)
    h_ref[...] = h
    g = jax.nn.sigmoid(jnp.dot(h, g1w_ref[...], preferred_element_type=_f32)
                       + g1b_ref[0, 0])
    g = g - jnp.max(g, axis=0, keepdims=True)
    p = jnp.exp(g)
    att0 = p / jnp.sum(p, axis=0, keepdims=True)
    hs0_ref[...] = lax.dot_general(att0, h, (((0,), (0,)), ((), ())),
                                   preferred_element_type=_f32)
    q1 = jnp.sum(wg_ref[...] * a1_ref[...][:, None, :], axis=-1)
    q2 = jnp.sum(wg_ref[...] * a2_ref[...][:, None, :], axis=-1)
    f1t_ref[...] = lax.dot_general(q1, h, (((1,), (1,)), ((), ())),
                                   preferred_element_type=_f32)
    f2t_ref[...] = lax.dot_general(q2, h, (((1,), (1,)), ((), ())),
                                   preferred_element_type=_f32)


def _prep(features, params):
    return pl.pallas_call(
        _prep_body,
        out_shape=[
            jax.ShapeDtypeStruct((N, HID), _f32),
            jax.ShapeDtypeStruct((1, HID), _f32),
            jax.ShapeDtypeStruct((NHEADS, N), _f32),
            jax.ShapeDtypeStruct((NHEADS, N), _f32),
        ],
    )(features, params['W_h'], params['a_h'], params['g1_w'],
      params['g1_b'].reshape(1, 1), params['Wg'], params['a1'], params['a2'])


def _attn1_body(h_ref, wg_ref, f1_ref, f2_ref, adj_ref, fe_ref, out_ref, whs):
    rb = pl.program_id(0)
    i = pl.program_id(1)

    @pl.when((rb == 0) & (i == 0))
    def _():
        hv = h_ref[...]
        for hh in range(NHEADS):
            whs[hh] = jnp.dot(hv, wg_ref[hh],
                              preferred_element_type=_f32)

    f1col = jnp.reshape(f1_ref[...], (BS1, 1))
    f2row = jnp.reshape(f2_ref[...], (1, N))
    z = f1col + f2row + fe_ref[0]
    z = jnp.where(adj_ref[...] > 0, _leaky(z), -1e9)
    z = z - jnp.max(z, axis=1, keepdims=True)
    p = jnp.exp(z)
    att = p / jnp.sum(p, axis=1, keepdims=True)
    wh_i = jnp.reshape(whs[pl.ds(i, 1)], (N, D2H))
    hp = jnp.dot(att, wh_i, preferred_element_type=_f32)
    out_ref[...] = _elu(hp)


def _attn1(h, wg, f1t, f2t, adjacency, fe):
    f1r = f1t.reshape(NHEADS, NBLK1, 1, BS1)
    f2r = f2t.reshape(NHEADS, 1, N)
    return pl.pallas_call(
        _attn1_body,
        grid=(NBLK1, NHEADS),
        in_specs=[
            pl.BlockSpec((N, HID), lambda rb, i: (0, 0)),
            pl.BlockSpec((NHEADS, HID, D2H), lambda rb, i: (0, 0, 0)),
            pl.BlockSpec((1, 1, 1, BS1), lambda rb, i: (i, rb, 0, 0)),
            pl.BlockSpec((1, 1, N), lambda rb, i: (i, 0, 0)),
            pl.BlockSpec((BS1, N), lambda rb, i: (rb, 0)),
            pl.BlockSpec((1, BS1, N), lambda rb, i: (i, rb, 0)),
        ],
        out_specs=pl.BlockSpec((BS1, D2H), lambda rb, i: (rb, i)),
        out_shape=jax.ShapeDtypeStruct((N, F1), _f32),
        scratch_shapes=[pltpu.VMEM((NHEADS, N, D2H), _f32)],
    )(h, wg, f1r, f2r, adjacency, fe)


def _mm_body(a_ref, b_ref, o_ref):
    k = pl.program_id(0)

    @pl.when(k == 0)
    def _():
        o_ref[...] = jnp.zeros_like(o_ref)

    o_ref[...] += jnp.dot(a_ref[...], b_ref[...], preferred_element_type=_f32)


def _matmul(a, b, bk):
    m, k = a.shape
    _, n = b.shape
    return pl.pallas_call(
        _mm_body,
        grid=(k // bk,),
        in_specs=[
            pl.BlockSpec((m, bk), lambda kb: (0, kb)),
            pl.BlockSpec((bk, n), lambda kb: (kb, 0)),
        ],
        out_specs=pl.BlockSpec((m, n), lambda kb: (0, 0)),
        out_shape=jax.ShapeDtypeStruct((m, n), _f32),
    )(a, b)


def _attn2_body(wh2_ref, wh2b_ref, ao1_ref, ao2_ref, adj_ref, fe_ref, out_ref):
    wh2 = wh2_ref[...]
    f2row = lax.dot_general(ao2_ref[...], wh2, (((1,), (1,)), ((), ())),
                            preferred_element_type=_f32)
    f1col = lax.dot_general(wh2b_ref[...], ao1_ref[...], (((1,), (1,)), ((), ())),
                            preferred_element_type=_f32)
    z = f1col + f2row + fe_ref[...]
    z = jnp.where(adj_ref[...] > 0, _leaky(z), -1e9)
    z = z - jnp.max(z, axis=1, keepdims=True)
    p = jnp.exp(z)
    att = p / jnp.sum(p, axis=1, keepdims=True)
    out_ref[...] = jnp.dot(att, wh2, preferred_element_type=_f32)


def _attn2(wh2, ao1, ao2, adj1, fe2):
    return pl.pallas_call(
        _attn2_body,
        grid=(NBLK2,),
        in_specs=[
            pl.BlockSpec((N1, D2H), lambda rb: (0, 0)),
            pl.BlockSpec((BS2, D2H), lambda rb: (rb, 0)),
            pl.BlockSpec((1, D2H), lambda rb: (0, 0)),
            pl.BlockSpec((1, D2H), lambda rb: (0, 0)),
            pl.BlockSpec((BS2, N1), lambda rb: (rb, 0)),
            pl.BlockSpec((BS2, N1), lambda rb: (rb, 0)),
        ],
        out_specs=pl.BlockSpec((BS2, D2H), lambda rb: (rb, 0)),
        out_shape=jax.ShapeDtypeStruct((N1, D2H), _f32),
    )(wh2, wh2, ao1.reshape(1, D2H), ao2.reshape(1, D2H), adj1, fe2)


@jax.jit
def kernel(features, edge_index, edge_attr, adjacency, node2node, params):
    src, dst = edge_index[0], edge_index[1]
    lin = src * N + dst

    h, hs0, f1t, f2t = _prep(features, params)

    fe = _fe_dense(params['a3'], node2node)

    h1 = _attn1(h, params['Wg'], f1t, f2t, adjacency, fe)

    ew = params['ep1_w']
    u = h1 @ ew[:F1, 0] + params['ep1_b'][0]
    v = h1 @ ew[F1:, 0]
    ngate = jax.nn.sigmoid(h1 @ params['ep1_ng_w'] + params['ep1_ng_b'])
    x1 = (h1 * ngate).reshape(N1, 2, F1).sum(axis=1)

    g2 = jax.nn.sigmoid(x1 @ params['g2_w'] + params['g2_b'])
    g2 = jax.nn.softmax(g2, axis=0)
    hs1 = jnp.sum(g2 * x1, axis=0, keepdims=True)

    we_cat = jnp.transpose(params['We'], (1, 0, 2)).reshape(DE, DE1)
    tpre = edge_attr @ (we_cat @ params['ao3'])
    fe2 = _sc_edge(edge_index, u, v, tpre)[:N1 * N1].reshape(N1, N1)
    s1, d1 = src // 2, dst // 2
    mask1 = s1 != d1
    adj1 = jnp.zeros((N1, N1), _f32).at[s1, d1].add(mask1.astype(_f32))

    wh2 = _matmul(x1, params['Wo'], 1024)
    h2 = _attn2(wh2, params['ao1'], params['ao2'], adj1, fe2)

    ngate2 = jax.nn.sigmoid(h2 @ params['ep2_ng_w'] + params['ep2_ng_b'])
    x2 = (h2 * ngate2).reshape(N1 // 2, 2, D2H).sum(axis=1)
    g3 = jax.nn.sigmoid(x2 @ params['g3_w'] + params['g3_b'])
    g3 = jax.nn.softmax(g3, axis=0)
    hs2 = jnp.sum(g3 * x2, axis=0, keepdims=True)

    return jnp.concatenate([hs0, hs1, hs2], axis=1)


# speedup vs baseline: 1.8593x; 1.0004x over previous
import functools

import jax
import jax.numpy as jnp
from jax import lax
from jax.experimental import pallas as pl
from jax.experimental.pallas import tpu as pltpu
from jax.experimental.pallas import tpu_sc as plsc

N = 2048
E = 65536
HID = 128
NHEADS = 16
DE = 16
DEH = 4
ALPHA = 0.2
D2H = 2 * HID
F1 = NHEADS * D2H
DE1 = NHEADS * DEH
N1 = N // 2

BS1 = 256
NBLK1 = N // BS1
BS2 = 256
NBLK2 = N1 // BS2

_f32 = jnp.float32


def _leaky(x):
    return jnp.where(x >= 0, x, ALPHA * x)


def _elu(x):
    return jnp.where(x > 0, x, jnp.exp(jnp.minimum(x, 0.0)) - 1.0)


NN = N * N
_SC_NC = 2
_SC_NS = 16
_NW = _SC_NC * _SC_NS
_CE = E // _NW
_i32 = jnp.int32


_SBD = 16


def _fed_body(a3_ref, n2n_ref, o_ref):
    t = lax.dot_general(a3_ref[...], n2n_ref[...],
                        (((1,), (1,)), ((), ())),
                        preferred_element_type=_f32)
    o_ref[...] = jnp.reshape(t, (NHEADS, _SBD, N))


def _fe_dense(a3, n2n):
    return pl.pallas_call(
        _fed_body,
        grid=(N // _SBD,),
        in_specs=[
            pl.BlockSpec((NHEADS, DE), lambda sb: (0, 0)),
            pl.BlockSpec((_SBD * N, DE), lambda sb: (sb, 0)),
        ],
        out_specs=pl.BlockSpec((NHEADS, _SBD, N), lambda sb: (0, sb, 0)),
        out_shape=jax.ShapeDtypeStruct((NHEADS, N, N), _f32),
    )(a3, n2n)


def _sc_edge_body(ei_hbm, u_hbm, v_hbm, tp_hbm, out_hbm,
                  src_v, dst_v, u_v, v_v, t_v, idx_v, val_v, sem):
    wid = lax.axis_index("s") * _SC_NC + lax.axis_index("c")
    base = wid * _CE
    pltpu.sync_copy(ei_hbm.at[0, pl.ds(base, _CE)], src_v)
    pltpu.sync_copy(ei_hbm.at[1, pl.ds(base, _CE)], dst_v)
    pltpu.sync_copy(u_hbm, u_v)
    pltpu.sync_copy(v_hbm, v_v)
    pltpu.sync_copy(tp_hbm.at[pl.ds(base, _CE)], t_v)

    def body(j, carry):
        sl = pl.ds(j * 16, 16)
        s = src_v[sl]
        d = dst_v[sl]
        ue = plsc.load_gather(u_v, [s])
        ve = plsc.load_gather(v_v, [d])
        score = 1.0 / (1.0 + jnp.exp(-(ue + ve)))
        s1 = lax.shift_right_logical(s, 1)
        d1 = lax.shift_right_logical(d, 1)
        idx_v[sl] = jnp.where(s1 != d1, s1 * N1 + d1, N1 * N1)
        val_v[sl] = score * t_v[sl]
        return carry

    lax.fori_loop(0, _CE // 16, body, 0)
    pltpu.async_copy(val_v, out_hbm.at[idx_v], sem).wait()


def _sc_edge(edge_index, u, v, tpre):
    run = pl.kernel(
        _sc_edge_body,
        mesh=plsc.VectorSubcoreMesh(core_axis_name="c", subcore_axis_name="s"),
        compiler_params=pltpu.CompilerParams(use_tc_tiling_on_sc=False,
                                             needs_layout_passes=False),
        out_type=jax.ShapeDtypeStruct((N1 * N1 + 8,), _f32),
        scratch_types=[
            pltpu.VMEM((_CE,), _i32),
            pltpu.VMEM((_CE,), _i32),
            pltpu.VMEM((N,), _f32),
            pltpu.VMEM((N,), _f32),
            pltpu.VMEM((_CE,), _f32),
            pltpu.VMEM((_CE,), _i32),
            pltpu.VMEM((_CE,), _f32),
            pltpu.SemaphoreType.DMA,
        ],
    )
    return run(edge_index, u, v, tpre)


def _prep_body(f_ref, wh_ref, ah_ref, g1w_ref, g1b_ref, wg_ref, a1_ref, a2_ref,
               h_ref, hs0_ref, f1t_ref, f2t_ref):
    f = f_ref[...]
    z = jnp.dot(f, wh_ref[...], preferred_element_type=_f32)
    gate = jax.nn.sigmoid(jnp.dot(f, ah_ref[...], preferred_element_type=_f32))
    h = _elu(z * gate)
    h_ref[...] = h
    g = jax.nn.sigmoid(jnp.dot(h, g1w_ref[...], preferred_element_type=_f32)
                       + g1b_ref[0, 0])
    g = g - jnp.max(g, axis=0, keepdims=True)
    p = jnp.exp(g)
    att0 = p / jnp.sum(p, axis=0, keepdims=True)
    hs0_ref[...] = lax.dot_general(att0, h, (((0,), (0,)), ((), ())),
                                   preferred_element_type=_f32)
    q1 = jnp.sum(wg_ref[...] * a1_ref[...][:, None, :], axis=-1)
    q2 = jnp.sum(wg_ref[...] * a2_ref[...][:, None, :], axis=-1)
    f1t_ref[...] = lax.dot_general(q1, h, (((1,), (1,)), ((), ())),
                                   preferred_element_type=_f32)
    f2t_ref[...] = lax.dot_general(q2, h, (((1,), (1,)), ((), ())),
                                   preferred_element_type=_f32)


def _prep(features, params):
    return pl.pallas_call(
        _prep_body,
        out_shape=[
            jax.ShapeDtypeStruct((N, HID), _f32),
            jax.ShapeDtypeStruct((1, HID), _f32),
            jax.ShapeDtypeStruct((NHEADS, N), _f32),
            jax.ShapeDtypeStruct((NHEADS, N), _f32),
        ],
    )(features, params['W_h'], params['a_h'], params['g1_w'],
      params['g1_b'].reshape(1, 1), params['Wg'], params['a1'], params['a2'])


def _attn1_body(h_ref, wg_ref, f1_ref, f2_ref, adj_ref, fe_ref, out_ref, whs):
    rb = pl.program_id(0)
    i = pl.program_id(1)

    @pl.when((rb == 0) & (i == 0))
    def _():
        hv = h_ref[...]
        for hh in range(NHEADS):
            whs[hh] = jnp.dot(hv, wg_ref[hh],
                              preferred_element_type=_f32)

    f1col = jnp.reshape(f1_ref[...], (BS1, 1))
    f2row = jnp.reshape(f2_ref[...], (1, N))
    z = f1col + f2row + fe_ref[0]
    z = jnp.where(adj_ref[...] > 0, _leaky(z), -1e9)
    z = z - jnp.max(z, axis=1, keepdims=True)
    p = jnp.exp(z)
    att = p / jnp.sum(p, axis=1, keepdims=True)
    wh_i = jnp.reshape(whs[pl.ds(i, 1)], (N, D2H))
    hp = jnp.dot(att, wh_i, preferred_element_type=_f32)
    out_ref[...] = _elu(hp)


def _attn1(h, wg, f1t, f2t, adjacency, fe):
    f1r = f1t.reshape(NHEADS, NBLK1, 1, BS1)
    f2r = f2t.reshape(NHEADS, 1, N)
    return pl.pallas_call(
        _attn1_body,
        grid=(NBLK1, NHEADS),
        in_specs=[
            pl.BlockSpec((N, HID), lambda rb, i: (0, 0)),
            pl.BlockSpec((NHEADS, HID, D2H), lambda rb, i: (0, 0, 0)),
            pl.BlockSpec((1, 1, 1, BS1), lambda rb, i: (i, rb, 0, 0)),
            pl.BlockSpec((1, 1, N), lambda rb, i: (i, 0, 0)),
            pl.BlockSpec((BS1, N), lambda rb, i: (rb, 0)),
            pl.BlockSpec((1, BS1, N), lambda rb, i: (i, rb, 0)),
        ],
        out_specs=pl.BlockSpec((BS1, D2H), lambda rb, i: (rb, i)),
        out_shape=jax.ShapeDtypeStruct((N, F1), _f32),
        scratch_shapes=[pltpu.VMEM((NHEADS, N, D2H), _f32)],
    )(h, wg, f1r, f2r, adjacency, fe)


def _mm_body(a_ref, b_ref, o_ref):
    k = pl.program_id(0)

    @pl.when(k == 0)
    def _():
        o_ref[...] = jnp.zeros_like(o_ref)

    o_ref[...] += jnp.dot(a_ref[...], b_ref[...], preferred_element_type=_f32)


def _matmul(a, b, bk):
    m, k = a.shape
    _, n = b.shape
    return pl.pallas_call(
        _mm_body,
        grid=(k // bk,),
        in_specs=[
            pl.BlockSpec((m, bk), lambda kb: (0, kb)),
            pl.BlockSpec((bk, n), lambda kb: (kb, 0)),
        ],
        out_specs=pl.BlockSpec((m, n), lambda kb: (0, 0)),
        out_shape=jax.ShapeDtypeStruct((m, n), _f32),
    )(a, b)


def _attn2_body(wh2_ref, wh2b_ref, ao1_ref, ao2_ref, adj_ref, fe_ref, out_ref):
    wh2 = wh2_ref[...]
    f2row = lax.dot_general(ao2_ref[...], wh2, (((1,), (1,)), ((), ())),
                            preferred_element_type=_f32)
    f1col = lax.dot_general(wh2b_ref[...], ao1_ref[...], (((1,), (1,)), ((), ())),
                            preferred_element_type=_f32)
    z = f1col + f2row + fe_ref[...]
    z = jnp.where(adj_ref[...] > 0, _leaky(z), -1e9)
    z = z - jnp.max(z, axis=1, keepdims=True)
    p = jnp.exp(z)
    att = p / jnp.sum(p, axis=1, keepdims=True)
    out_ref[...] = jnp.dot(att, wh2, preferred_element_type=_f32)


def _attn2(wh2, ao1, ao2, adj1, fe2):
    return pl.pallas_call(
        _attn2_body,
        grid=(NBLK2,),
        in_specs=[
            pl.BlockSpec((N1, D2H), lambda rb: (0, 0)),
            pl.BlockSpec((BS2, D2H), lambda rb: (rb, 0)),
            pl.BlockSpec((1, D2H), lambda rb: (0, 0)),
            pl.BlockSpec((1, D2H), lambda rb: (0, 0)),
            pl.BlockSpec((BS2, N1), lambda rb: (rb, 0)),
            pl.BlockSpec((BS2, N1), lambda rb: (rb, 0)),
        ],
        out_specs=pl.BlockSpec((BS2, D2H), lambda rb: (rb, 0)),
        out_shape=jax.ShapeDtypeStruct((N1, D2H), _f32),
    )(wh2, wh2, ao1.reshape(1, D2H), ao2.reshape(1, D2H), adj1, fe2)


@jax.jit
def kernel(features, edge_index, edge_attr, adjacency, node2node, params):
    src, dst = edge_index[0], edge_index[1]
    lin = src * N + dst

    h, hs0, f1t, f2t = _prep(features, params)

    fe = _fe_dense(params['a3'], node2node)

    h1 = _attn1(h, params['Wg'], f1t, f2t, adjacency, fe)

    ew = params['ep1_w']
    u = h1 @ ew[:F1, 0] + params['ep1_b'][0]
    v = h1 @ ew[F1:, 0]
    ngate = jax.nn.sigmoid(h1 @ params['ep1_ng_w'] + params['ep1_ng_b'])
    x1 = (h1 * ngate).reshape(N1, 2, F1).sum(axis=1)

    g2 = jax.nn.sigmoid(x1 @ params['g2_w'] + params['g2_b'])
    g2 = jax.nn.softmax(g2, axis=0)
    hs1 = jnp.sum(g2 * x1, axis=0, keepdims=True)

    we_cat = jnp.transpose(params['We'], (1, 0, 2)).reshape(DE, DE1)
    tpre = edge_attr @ (we_cat @ params['ao3'])
    fe2 = _sc_edge(edge_index, u, v, tpre)[:N1 * N1].reshape(N1, N1)
    s1, d1 = src // 2, dst // 2
    mask1 = s1 != d1
    adj1 = jnp.zeros((N1, N1), _f32).at[s1, d1].add(mask1.astype(_f32))

    wh2 = _matmul(x1, params['Wo'], 1024)
    h2 = _attn2(wh2, params['ao1'], params['ao2'], adj1, fe2)

    ngate2 = jax.nn.sigmoid(h2 @ params['ep2_ng_w'] + params['ep2_ng_b'])
    x2 = (h2 * ngate2).reshape(N1 // 2, 2, D2H).sum(axis=1)
    g3 = jax.nn.sigmoid(x2 @ params['g3_w'] + params['g3_b'])
    g3 = jax.nn.softmax(g3, axis=0)
    hs2 = jnp.sum(g3 * x2, axis=0, keepdims=True)

    return jnp.concatenate([hs0, hs1, hs2], axis=1)
